# Initial kernel scaffold; baseline (speedup 1.0000x reference)
import jax
import jax.numpy as jnp
from jax.experimental import pallas as pl


def kernel(embed, prefix_sum, W1, b1, W2, b2):
    raise NotImplementedError("write your pallas kernel here")


# speedup vs baseline: 5.1111x; 5.1111x over previous
import functools

import jax
import jax.numpy as jnp
from jax.experimental import pallas as pl
from jax.experimental.pallas import tpu as pltpu

B = 16
N = 32768
D = 128
H = 256
ROWS = 2048
NBLK = N // ROWS


def _fused_body(pf_ref, x_ref, w1_ref, b1_ref, w2_ref, b2_ref, out_ref,
                segacc_ref, gb_ref):
    phase = pl.program_id(0)
    j = pl.program_id(1)

    rows = j * ROWS + jax.lax.broadcasted_iota(jnp.int32, (ROWS, 1), 0)
    seg = jnp.sum((rows >= pf_ref[...]).astype(jnp.int32), axis=1,
                  keepdims=True)
    bidx = jax.lax.broadcasted_iota(jnp.int32, (1, B), 1)
    onehot = (seg == bidx).astype(jnp.float32)

    @pl.when(phase == 0)
    def _():
        part = jax.lax.dot_general(
            onehot, x_ref[...], (((0,), (0,)), ((), ())),
            preferred_element_type=jnp.float32)

        @pl.when(j == 0)
        def _():
            segacc_ref[...] = part

        @pl.when(j != 0)
        def _():
            segacc_ref[...] = segacc_ref[...] + part

        @pl.when(j == NBLK - 1)
        def _():
            gb_ref[...] = jnp.dot(
                segacc_ref[...], w1_ref[D:, :],
                preferred_element_type=jnp.float32) + b1_ref[...]

    @pl.when(phase == 1)
    def _():
        bias = jnp.dot(onehot, gb_ref[...],
                       preferred_element_type=jnp.float32)
        h = jnp.maximum(
            jnp.dot(x_ref[...], w1_ref[:D, :],
                    preferred_element_type=jnp.float32) + bias, 0.0)
        out_ref[...] = jnp.dot(h, w2_ref[...],
                               preferred_element_type=jnp.float32) + b2_ref[...]


@jax.jit
def _run(embed, prefix_sum, W1, b1, W2, b2):
    pf2 = prefix_sum.reshape(1, B)
    out = pl.pallas_call(
        _fused_body,
        grid=(2, NBLK),
        in_specs=[
            pl.BlockSpec((1, B), lambda i, j: (0, 0)),
            pl.BlockSpec((ROWS, D), lambda i, j: (j, 0)),
            pl.BlockSpec((2 * D, H), lambda i, j: (0, 0)),
            pl.BlockSpec((1, H), lambda i, j: (0, 0)),
            pl.BlockSpec((H, 1), lambda i, j: (0, 0)),
            pl.BlockSpec((1, 1), lambda i, j: (0, 0)),
        ],
        out_specs=pl.BlockSpec((ROWS, 1), lambda i, j: (j, 0)),
        out_shape=jax.ShapeDtypeStruct((N, 1), jnp.float32),
        scratch_shapes=[
            pltpu.VMEM((B, D), jnp.float32),
            pltpu.VMEM((B, H), jnp.float32),
        ],
        compiler_params=pltpu.CompilerParams(
            dimension_semantics=("arbitrary", "arbitrary")),
    )(pf2, embed, W1, b1.reshape(1, H), W2, b2.reshape(1, 1))
    return out


def kernel(embed, prefix_sum, W1, b1, W2, b2):
    return (_run(embed, prefix_sum, W1, b1, W2, b2), prefix_sum)


# speedup vs baseline: 6.1648x; 1.2062x over previous
import jax
import jax.numpy as jnp
from jax.experimental import pallas as pl
from jax.experimental.pallas import tpu as pltpu

B = 16
N = 32768
D = 128
H = 256
ROWS = 4096
NBLK = N // ROWS


def _fused_body(pf_ref, pfs_ref, x_ref, w1_ref, b1_ref, w2_ref, b2_ref,
                out_ref, segacc_ref, gb_ref):
    phase = pl.program_id(0)
    j = pl.program_id(1)

    rows = j * ROWS + jax.lax.broadcasted_iota(jnp.int32, (ROWS, 1), 0)
    lo = (rows >= pfs_ref[...]).astype(jnp.float32)
    hi = (rows >= pf_ref[...]).astype(jnp.float32)
    onehot = lo - hi

    @pl.when(phase == 0)
    def _():
        part = jax.lax.dot_general(
            onehot, x_ref[...], (((0,), (0,)), ((), ())),
            preferred_element_type=jnp.float32)

        @pl.when(j == 0)
        def _():
            segacc_ref[...] = part

        @pl.when(j != 0)
        def _():
            segacc_ref[...] = segacc_ref[...] + part

        @pl.when(j == NBLK - 1)
        def _():
            gb_ref[...] = jnp.dot(
                segacc_ref[...], w1_ref[D:, :],
                preferred_element_type=jnp.float32) + b1_ref[...]

    @pl.when(phase == 1)
    def _():
        bias = jnp.dot(onehot, gb_ref[...],
                       preferred_element_type=jnp.float32)
        h = jnp.maximum(
            jnp.dot(x_ref[...], w1_ref[:D, :],
                    preferred_element_type=jnp.float32) + bias, 0.0)
        out_ref[...] = jnp.dot(h, w2_ref[...],
                               preferred_element_type=jnp.float32) + b2_ref[...]


@jax.jit
def _run(embed, prefix_sum, W1, b1, W2, b2):
    pf2 = prefix_sum.reshape(1, B)
    pfs2 = jnp.concatenate(
        [jnp.zeros((1, 1), prefix_sum.dtype), pf2[:, : B - 1]], axis=1)
    out = pl.pallas_call(
        _fused_body,
        grid=(2, NBLK),
        in_specs=[
            pl.BlockSpec((1, B), lambda i, j: (0, 0)),
            pl.BlockSpec((1, B), lambda i, j: (0, 0)),
            pl.BlockSpec((ROWS, D), lambda i, j: (j, 0)),
            pl.BlockSpec((2 * D, H), lambda i, j: (0, 0)),
            pl.BlockSpec((1, H), lambda i, j: (0, 0)),
            pl.BlockSpec((H, 1), lambda i, j: (0, 0)),
            pl.BlockSpec((1, 1), lambda i, j: (0, 0)),
        ],
        out_specs=pl.BlockSpec((ROWS, 1), lambda i, j: (j, 0)),
        out_shape=jax.ShapeDtypeStruct((N, 1), jnp.float32),
        scratch_shapes=[
            pltpu.VMEM((B, D), jnp.float32),
            pltpu.VMEM((B, H), jnp.float32),
        ],
        compiler_params=pltpu.CompilerParams(
            dimension_semantics=("arbitrary", "arbitrary")),
    )(pf2, pfs2, embed, W1, b1.reshape(1, H), W2, b2.reshape(1, 1))
    return out


def kernel(embed, prefix_sum, W1, b1, W2, b2):
    return (_run(embed, prefix_sum, W1, b1, W2, b2), prefix_sum)


# speedup vs baseline: 6.6176x; 1.0734x over previous
import jax
import jax.numpy as jnp
from jax.experimental import pallas as pl
from jax.experimental.pallas import tpu as pltpu

B = 16
N = 32768
D = 128
H = 256
ROWS = 4096
NBLK = N // ROWS


def _fused_body(pf_ref, pfs_ref, x_ref, w1_ref, b1_ref, w2_ref, b2_ref,
                out_ref, segacc_ref, gb_ref):
    phase = pl.program_id(0)
    j = pl.program_id(1)

    rows = j * ROWS + jax.lax.broadcasted_iota(jnp.int32, (ROWS, 1), 0)
    lo = (rows >= pfs_ref[...]).astype(jnp.float32)
    hi = (rows >= pf_ref[...]).astype(jnp.float32)
    onehot = lo - hi

    @pl.when(phase == 0)
    def _():
        part = jax.lax.dot_general(
            onehot, x_ref[...], (((0,), (0,)), ((), ())),
            preferred_element_type=jnp.float32)

        @pl.when(j == 0)
        def _():
            segacc_ref[...] = part

        @pl.when(j != 0)
        def _():
            segacc_ref[...] = segacc_ref[...] + part

        @pl.when(j == NBLK - 1)
        def _():
            gb_ref[...] = jnp.dot(
                segacc_ref[...], w1_ref[D:, :],
                preferred_element_type=jnp.float32) + b1_ref[...]

    @pl.when(phase == 1)
    def _():
        bias = jnp.dot(onehot, gb_ref[...],
                       preferred_element_type=jnp.float32)
        h = jnp.maximum(
            jnp.dot(x_ref[...], w1_ref[:D, :],
                    preferred_element_type=jnp.float32) + bias, 0.0)
        out_ref[...] = jnp.dot(h, w2_ref[...],
                               preferred_element_type=jnp.float32) + b2_ref[...]


@jax.jit
def _run(embed, prefix_sum, W1, b1, W2, b2):
    pf2 = prefix_sum.reshape(1, B)
    pfs2 = jnp.concatenate(
        [jnp.zeros((1, 1), prefix_sum.dtype), pf2[:, : B - 1]], axis=1)
    out = pl.pallas_call(
        _fused_body,
        grid=(2, NBLK),
        in_specs=[
            pl.BlockSpec((1, B), lambda i, j: (0, 0)),
            pl.BlockSpec((1, B), lambda i, j: (0, 0)),
            pl.BlockSpec((ROWS, D), lambda i, j: (j, 0)),
            pl.BlockSpec((2 * D, H), lambda i, j: (0, 0)),
            pl.BlockSpec((1, H), lambda i, j: (0, 0)),
            pl.BlockSpec((H, 1), lambda i, j: (0, 0)),
            pl.BlockSpec((1, 1), lambda i, j: (0, 0)),
        ],
        out_specs=pl.BlockSpec((ROWS, 1), lambda i, j: (i * j, 0)),
        out_shape=jax.ShapeDtypeStruct((N, 1), jnp.float32),
        scratch_shapes=[
            pltpu.VMEM((B, D), jnp.float32),
            pltpu.VMEM((B, H), jnp.float32),
        ],
        compiler_params=pltpu.CompilerParams(
            dimension_semantics=("arbitrary", "arbitrary")),
    )(pf2, pfs2, embed, W1, b1.reshape(1, H), W2, b2.reshape(1, 1))
    return out


def kernel(embed, prefix_sum, W1, b1, W2, b2):
    return (_run(embed, prefix_sum, W1, b1, W2, b2), prefix_sum)


# speedup vs baseline: 7.8556x; 1.1871x over previous
import jax
import jax.numpy as jnp
from jax.experimental import pallas as pl
from jax.experimental.pallas import tpu as pltpu

B = 16
N = 32768
D = 128
H = 256
ROWS_P = 8192
ROWS_M = 4096
NBLK_P = N // ROWS_P
NBLK_M = N // ROWS_M


def _pool_body(pfs_ref, x_ref, w1_ref, b1_ref, dg_ref, sacc_ref):
    j = pl.program_id(0)
    rows = j * ROWS_P + jax.lax.broadcasted_iota(jnp.int32, (ROWS_P, 1), 0)
    lo = (rows >= pfs_ref[...]).astype(jnp.float32)
    part = jax.lax.dot_general(
        lo, x_ref[...], (((0,), (0,)), ((), ())),
        preferred_element_type=jnp.float32)

    @pl.when(j == 0)
    def _():
        sacc_ref[...] = part

    @pl.when(j != 0)
    def _():
        sacc_ref[...] = sacc_ref[...] + part

    @pl.when(j == NBLK_P - 1)
    def _():
        s = sacc_ref[...]
        ge = s - jnp.concatenate([s[1:], jnp.zeros((1, D), jnp.float32)], 0)
        g = jnp.dot(ge, w1_ref[D:, :],
                    preferred_element_type=jnp.float32) + b1_ref[...]
        dg_ref[...] = g - jnp.concatenate(
            [jnp.zeros((1, H), jnp.float32), g[: B - 1]], 0)


def _mlp_body(pfs_ref, x_ref, w1_ref, dg_ref, w2_ref, b2_ref, out_ref):
    j = pl.program_id(0)
    rows = j * ROWS_M + jax.lax.broadcasted_iota(jnp.int32, (ROWS_M, 1), 0)
    lo = (rows >= pfs_ref[...]).astype(jnp.float32)
    bias = jnp.dot(lo, dg_ref[...],
                   preferred_element_type=jnp.float32)
    h = jnp.maximum(
        jnp.dot(x_ref[...], w1_ref[:D, :],
                preferred_element_type=jnp.float32) + bias, 0.0)
    out_ref[...] = jnp.dot(h, w2_ref[...],
                           preferred_element_type=jnp.float32) + b2_ref[...]


@jax.jit
def _run(embed, prefix_sum, W1, b1, W2, b2):
    pfs = jnp.concatenate(
        [jnp.zeros((1, 1), prefix_sum.dtype),
         prefix_sum.reshape(1, B)[:, : B - 1]], axis=1)
    dg = pl.pallas_call(
        _pool_body,
        grid=(NBLK_P,),
        in_specs=[
            pl.BlockSpec((1, B), lambda j: (0, 0)),
            pl.BlockSpec((ROWS_P, D), lambda j: (j, 0)),
            pl.BlockSpec((2 * D, H), lambda j: (0, 0)),
            pl.BlockSpec((1, H), lambda j: (0, 0)),
        ],
        out_specs=pl.BlockSpec((B, H), lambda j: (0, 0)),
        out_shape=jax.ShapeDtypeStruct((B, H), jnp.float32),
        scratch_shapes=[pltpu.VMEM((B, D), jnp.float32)],
        compiler_params=pltpu.CompilerParams(
            dimension_semantics=("arbitrary",)),
    )(pfs, embed, W1, b1.reshape(1, H))
    out = pl.pallas_call(
        _mlp_body,
        grid=(NBLK_M,),
        in_specs=[
            pl.BlockSpec((1, B), lambda j: (0, 0)),
            pl.BlockSpec((ROWS_M, D), lambda j: (j, 0)),
            pl.BlockSpec((2 * D, H), lambda j: (0, 0)),
            pl.BlockSpec((B, H), lambda j: (0, 0)),
            pl.BlockSpec((H, 1), lambda j: (0, 0)),
            pl.BlockSpec((1, 1), lambda j: (0, 0)),
        ],
        out_specs=pl.BlockSpec((ROWS_M, 1), lambda j: (j, 0)),
        out_shape=jax.ShapeDtypeStruct((N, 1), jnp.float32),
        compiler_params=pltpu.CompilerParams(
            dimension_semantics=("arbitrary",)),
    )(pfs, embed, W1, dg, W2, b2.reshape(1, 1))
    return out


def kernel(embed, prefix_sum, W1, b1, W2, b2):
    return (_run(embed, prefix_sum, W1, b1, W2, b2), prefix_sum)
